# baseline (speedup 1.0000x reference)
import functools

import jax
import jax.numpy as jnp
from jax import lax
from jax.experimental import pallas as pl
from jax.experimental.pallas import tpu as pltpu
from jax.experimental.pallas import tpu_sc as plsc

_VOCAB = 100000
_HIDDEN = 768
_BATCH = 4
_SEQ = 2048
_B = _BATCH * _SEQ

_NC = 2
_NS = 16
_NW = _NC * _NS
_B_PER_W = _B // _NW
_CHUNK = 64
_N_CHUNKS = _B_PER_W // _CHUNK
_L = 16
_NG = _HIDDEN // _L


def _emb_body(src_hbm, seg_hbm, tok_tab_hbm, seg_tab_hbm, pos_hbm, out_hbm,
              idx_v, seg_v, seg_tab_v, buf_v, pos_v, sem):
    wid = lax.axis_index("s") * _NC + lax.axis_index("c")
    base = pl.multiple_of(wid * _B_PER_W, _B_PER_W)
    pltpu.sync_copy(seg_tab_hbm, seg_tab_v)
    for c in range(_N_CHUNKS):
        off = base + c * _CHUNK
        pos_off = lax.rem(off, _SEQ)
        pltpu.sync_copy(src_hbm.at[pl.ds(off, _CHUNK)], idx_v)
        pltpu.sync_copy(seg_hbm.at[pl.ds(off, _CHUNK)], seg_v)
        cp = pltpu.async_copy(tok_tab_hbm.at[idx_v], buf_v, sem)
        pltpu.sync_copy(pos_hbm.at[pl.ds(pos_off, _CHUNK)], pos_v)
        cp.wait()

        def g_body(g, carry):
            goff = pl.multiple_of(g * _L, _L)
            s0 = seg_tab_v[0, pl.ds(goff, _L)]
            s1 = seg_tab_v[1, pl.ds(goff, _L)]
            d = s1 - s0

            def blk_body(rb, inner):
                segf16 = seg_v[pl.ds(rb * _L, _L)].astype(jnp.float32)
                for j in range(_L):
                    r = rb * _L + j
                    segf = jnp.full((_L,), segf16[j])
                    p = pos_v[r, pl.ds(goff, _L)]
                    plsc.addupdate(buf_v.at[r, pl.ds(goff, _L)],
                                   p + s0 + segf * d)
                return inner

            lax.fori_loop(0, _CHUNK // _L, blk_body, 0)
            return carry

        lax.fori_loop(0, _NG, g_body, 0)

        pltpu.sync_copy(buf_v, out_hbm.at[pl.ds(off, _CHUNK)])


@jax.jit
def _emb(src_flat, seg_flat, token_table, segment_table, pos_flat):
    mesh = plsc.VectorSubcoreMesh(
        core_axis_name="c", subcore_axis_name="s",
        num_cores=_NC, num_subcores=_NS)
    run = functools.partial(
        pl.kernel,
        out_type=jax.ShapeDtypeStruct((_B, _HIDDEN), jnp.float32),
        mesh=mesh,
        scratch_types=[
            pltpu.VMEM((_CHUNK,), jnp.int32),
            pltpu.VMEM((_CHUNK,), jnp.int32),
            pltpu.VMEM((2, _HIDDEN), jnp.float32),
            pltpu.VMEM((_CHUNK, _HIDDEN), jnp.float32),
            pltpu.VMEM((_CHUNK, _HIDDEN), jnp.float32),
            pltpu.SemaphoreType.DMA,
        ],
    )(_emb_body)
    return run(src_flat, seg_flat, token_table, segment_table, pos_flat)


def kernel(source, segment, token_table, segment_table, pos_emb):
    src_flat = source.reshape(_B).astype(jnp.int32)
    seg_flat = segment.reshape(_B).astype(jnp.int32)
    pos_flat = pos_emb.reshape(pos_emb.shape[1], _HIDDEN)
    out = _emb(src_flat, seg_flat, token_table, segment_table, pos_flat)
    return out.reshape(_BATCH, _SEQ, _HIDDEN)


# speedup vs baseline: 1.4146x; 1.4146x over previous
import functools

import jax
import jax.numpy as jnp
from jax import lax
from jax.experimental import pallas as pl
from jax.experimental.pallas import tpu as pltpu
from jax.experimental.pallas import tpu_sc as plsc

_VOCAB = 100000
_HIDDEN = 768
_BATCH = 4
_SEQ = 2048
_B = _BATCH * _SEQ

_NC = 2
_NS = 16
_NW = _NC * _NS
_S_PER_W = _SEQ // _NW
_CHUNK = 32
_L = 16
_NG = _HIDDEN // _L


def _emb_body(src_hbm, seg_hbm, tok_tab_hbm, seg_tab_hbm, pos_hbm, out_hbm,
              idx0, idx1, sg0, sg1, seg_tab_v, pos_v, buf0, buf1, sem0, sem1):
    idx = [idx0, idx1]
    segv = [sg0, sg1]
    buf = [buf0, buf1]
    sem = [sem0, sem1]
    wid = lax.axis_index("s") * _NC + lax.axis_index("c")
    s_base = pl.multiple_of(wid * _S_PER_W, _S_PER_W)
    pltpu.sync_copy(seg_tab_hbm, seg_tab_v)
    pltpu.sync_copy(pos_hbm.at[pl.ds(s_base, _S_PER_W)], pos_v)

    chunks = [(t, b) for t in range(_S_PER_W // _CHUNK) for b in range(_BATCH)]
    n = len(chunks)

    def flat_off(t, b):
        return pl.multiple_of(b * _SEQ + s_base + t * _CHUNK, _CHUNK)

    def add_posseg(buf_ref, segv_ref, pos_base):
        def g_body(g, carry):
            goff = pl.multiple_of(g * _L, _L)
            s0 = seg_tab_v[0, pl.ds(goff, _L)]
            s1 = seg_tab_v[1, pl.ds(goff, _L)]
            d = s1 - s0

            def blk_body(rb, inner):
                segf16 = segv_ref[pl.ds(rb * _L, _L)].astype(jnp.float32)
                for j in range(_L):
                    r = rb * _L + j
                    segf = jnp.full((_L,), segf16[j])
                    p = pos_v[pos_base + r, pl.ds(goff, _L)]
                    plsc.addupdate(buf_ref.at[r, pl.ds(goff, _L)],
                                   p + s0 + segf * d)
                return inner

            lax.fori_loop(0, _CHUNK // _L, blk_body, 0)
            return carry

        lax.fori_loop(0, _NG, g_body, 0)

    cps = [None, None]
    t0, b0 = chunks[0]
    off0 = flat_off(t0, b0)
    pltpu.sync_copy(src_hbm.at[pl.ds(off0, _CHUNK)], idx[0])
    cps[0] = pltpu.async_copy(tok_tab_hbm.at[idx[0]], buf[0], sem[0])
    pltpu.sync_copy(seg_hbm.at[pl.ds(off0, _CHUNK)], segv[0])

    for i, (t, b) in enumerate(chunks):
        cur = i % 2
        nxt = (i + 1) % 2
        if i + 1 < n:
            tn, bn = chunks[i + 1]
            offn = flat_off(tn, bn)
            pltpu.sync_copy(src_hbm.at[pl.ds(offn, _CHUNK)], idx[nxt])
            cps[nxt] = pltpu.async_copy(
                tok_tab_hbm.at[idx[nxt]], buf[nxt], sem[nxt])
            pltpu.sync_copy(seg_hbm.at[pl.ds(offn, _CHUNK)], segv[nxt])
        cps[cur].wait()
        add_posseg(buf[cur], segv[cur], t * _CHUNK)
        pltpu.sync_copy(buf[cur], out_hbm.at[pl.ds(flat_off(t, b), _CHUNK)])


@jax.jit
def _emb(src_flat, seg_flat, token_table, segment_table, pos_flat):
    mesh = plsc.VectorSubcoreMesh(
        core_axis_name="c", subcore_axis_name="s",
        num_cores=_NC, num_subcores=_NS)
    run = functools.partial(
        pl.kernel,
        out_type=jax.ShapeDtypeStruct((_B, _HIDDEN), jnp.float32),
        mesh=mesh,
        scratch_types=[
            pltpu.VMEM((_CHUNK,), jnp.int32),
            pltpu.VMEM((_CHUNK,), jnp.int32),
            pltpu.VMEM((_CHUNK,), jnp.int32),
            pltpu.VMEM((_CHUNK,), jnp.int32),
            pltpu.VMEM((2, _HIDDEN), jnp.float32),
            pltpu.VMEM((_S_PER_W, _HIDDEN), jnp.float32),
            pltpu.VMEM((_CHUNK, _HIDDEN), jnp.float32),
            pltpu.VMEM((_CHUNK, _HIDDEN), jnp.float32),
            pltpu.SemaphoreType.DMA,
            pltpu.SemaphoreType.DMA,
        ],
    )(_emb_body)
    return run(src_flat, seg_flat, token_table, segment_table, pos_flat)


def kernel(source, segment, token_table, segment_table, pos_emb):
    src_flat = source.reshape(_B).astype(jnp.int32)
    seg_flat = segment.reshape(_B).astype(jnp.int32)
    pos_flat = pos_emb.reshape(pos_emb.shape[1], _HIDDEN)
    out = _emb(src_flat, seg_flat, token_table, segment_table, pos_flat)
    return out.reshape(_BATCH, _SEQ, _HIDDEN)
